# Initial kernel scaffold; baseline (speedup 1.0000x reference)
import jax
import jax.numpy as jnp
from jax.experimental import pallas as pl


def kernel(x, h_a, h_t, W_moe, b_moe, norm_g, norm_b, gate, W_in, b_in, W_out, b_out, expert_idx):
    raise NotImplementedError("write your pallas kernel here")


# speedup vs baseline: 1.5943x; 1.5943x over previous
import functools
import math

import jax
import jax.numpy as jnp
import numpy as np
from jax.experimental import pallas as pl
from jax.experimental.pallas import tpu as pltpu

_B, _T, _K = 64, 64, 32
_IN_DIM, _HID, _ACT = 4096, 1024, 7
_NE, _NH, _NB = 8, 8, 2
_HD = _HID // _NH

_BF = jnp.bfloat16
_F32 = jnp.float32


def _rope_tables(seq_len, dim):
    inv_freq = 1.0 / (10000.0 ** (jnp.arange(0, dim, 2, dtype=_F32) / dim))
    t = jnp.arange(seq_len, dtype=_F32)
    freqs = t[:, None] * inv_freq[None, :]
    emb = jnp.concatenate([freqs, freqs], axis=-1)
    return jnp.cos(emb), jnp.sin(emb)


def _rope_coeff(seq_len, dim, reps):
    cos, sin = _rope_tables(seq_len, dim)
    lane = jnp.arange(dim)
    even = (lane % 2) == 0
    a = jnp.where(even[None, :], -sin, 0.0)
    b = jnp.where(even[None, :], 0.0, sin)
    tile = lambda z: jnp.tile(z, (1, reps))
    return tile(cos), tile(a), tile(b)


def _roll_l(z):
    return jnp.concatenate([z[:, 1:], z[:, :1]], axis=1)


def _roll_r(z):
    return jnp.concatenate([z[:, -1:], z[:, :-1]], axis=1)


def _mm_kernel(x_ref, w_ref, b_ref, o_ref):
    k = pl.program_id(1)

    @pl.when(k == 0)
    def _():
        o_ref[...] = jnp.zeros_like(o_ref)

    o_ref[...] += jnp.dot(x_ref[...].astype(_BF), w_ref[...].astype(_BF),
                          preferred_element_type=_F32)

    @pl.when(k == pl.num_programs(1) - 1)
    def _():
        o_ref[...] += b_ref[...]


def _matmul_bias(x, w, b_row, mb, kb):
    m, kk = x.shape
    n = w.shape[1]
    return pl.pallas_call(
        _mm_kernel,
        grid=(m // mb, kk // kb),
        in_specs=[
            pl.BlockSpec((mb, kb), lambda i, k: (i, k)),
            pl.BlockSpec((kb, n), lambda i, k: (k, 0)),
            pl.BlockSpec((1, n), lambda i, k: (0, 0)),
        ],
        out_specs=pl.BlockSpec((mb, n), lambda i, k: (i, 0)),
        out_shape=jax.ShapeDtypeStruct((m, n), _F32),
        compiler_params=pltpu.CompilerParams(
            dimension_semantics=("parallel", "arbitrary")),
    )(x, w, b_row)


def _dual_kernel(e_ref, x_ref, w1_ref, w2_ref, b1_ref, b2_ref, o_ref):
    k = pl.program_id(1)

    @pl.when(k == 0)
    def _():
        o_ref[...] = jnp.zeros_like(o_ref)

    xb = x_ref[...].astype(_BF)
    o_ref[:, :_HID] += jnp.dot(xb, w1_ref[...].astype(_BF),
                               preferred_element_type=_F32)
    o_ref[:, _HID:] += jnp.dot(xb, w2_ref[...].astype(_BF),
                               preferred_element_type=_F32)

    @pl.when(k == pl.num_programs(1) - 1)
    def _():
        o_ref[:, :_HID] += b1_ref[...]
        o_ref[:, _HID:] += b2_ref[...]


def _expert_dual(e_arr, x, w_moe, b_moe, layer, i1, i2, mb, kb):
    m = x.shape[0]
    wspec = lambda i: pl.BlockSpec(
        (None, None, None, kb, _HID),
        lambda mi, k, e: (layer, i, e[0], k, 0))
    bspec = lambda i: pl.BlockSpec(
        (None, 1, _HID), lambda mi, k, e: (layer * 8 * _NE + i * _NE + e[0], 0, 0))
    return pl.pallas_call(
        _dual_kernel,
        grid_spec=pltpu.PrefetchScalarGridSpec(
            num_scalar_prefetch=1,
            grid=(m // mb, _HID // kb),
            in_specs=[
                pl.BlockSpec((mb, kb), lambda mi, k, e: (mi, k)),
                wspec(i1), wspec(i2), bspec(i1), bspec(i2),
            ],
            out_specs=pl.BlockSpec((mb, 2 * _HID), lambda mi, k, e: (mi, 0)),
        ),
        out_shape=jax.ShapeDtypeStruct((m, 2 * _HID), _F32),
        compiler_params=pltpu.CompilerParams(
            dimension_semantics=("parallel", "arbitrary")),
    )(e_arr, x, w_moe, w_moe, b_moe, b_moe)


def _oln_kernel(e_ref, a_ref, w_ref, b_ref, res_ref, g_ref, be_ref,
                o_ref, acc_ref):
    k = pl.program_id(1)

    @pl.when(k == 0)
    def _():
        acc_ref[...] = jnp.zeros_like(acc_ref)

    acc_ref[...] += jnp.dot(a_ref[...].astype(_BF), w_ref[...].astype(_BF),
                            preferred_element_type=_F32)

    @pl.when(k == pl.num_programs(1) - 1)
    def _():
        y = acc_ref[...] + b_ref[...] + res_ref[...]
        mu = jnp.mean(y, axis=1, keepdims=True)
        d = y - mu
        var = jnp.mean(d * d, axis=1, keepdims=True)
        o_ref[...] = d * jax.lax.rsqrt(var + 1e-5) * g_ref[...] + be_ref[...]


def _expert_oln(e_arr, attn, w_moe, b_moe, res, norm_g, norm_b, layer, mb, kb):
    m = attn.shape[0]
    return pl.pallas_call(
        _oln_kernel,
        grid_spec=pltpu.PrefetchScalarGridSpec(
            num_scalar_prefetch=1,
            grid=(m // mb, _HID // kb),
            in_specs=[
                pl.BlockSpec((mb, kb), lambda mi, k, e: (mi, k)),
                pl.BlockSpec((None, None, None, kb, _HID),
                             lambda mi, k, e: (layer, 6, e[0], k, 0)),
                pl.BlockSpec((None, 1, _HID),
                             lambda mi, k, e: (layer * 8 * _NE + 6 * _NE + e[0], 0, 0)),
                pl.BlockSpec((mb, _HID), lambda mi, k, e: (mi, 0)),
                pl.BlockSpec((None, 1, _HID),
                             lambda mi, k, e: (layer * _NE + e[0], 0, 0)),
                pl.BlockSpec((None, 1, _HID),
                             lambda mi, k, e: (layer * _NE + e[0], 0, 0)),
            ],
            out_specs=pl.BlockSpec((mb, _HID), lambda mi, k, e: (mi, 0)),
            scratch_shapes=[pltpu.VMEM((mb, _HID), _F32)],
        ),
        out_shape=jax.ShapeDtypeStruct((m, _HID), _F32),
        compiler_params=pltpu.CompilerParams(
            dimension_semantics=("parallel", "arbitrary")),
    )(e_arr, attn, w_moe, b_moe, res, norm_g, norm_b)


def _ffn_kernel(e_ref, x_ref, w_ref, b_ref, o_ref):
    k = pl.program_id(1)

    @pl.when(k == 0)
    def _():
        o_ref[...] = jnp.zeros_like(o_ref)

    o_ref[...] += jnp.dot(x_ref[...].astype(_BF), w_ref[...].astype(_BF),
                          preferred_element_type=_F32)

    @pl.when(k == pl.num_programs(1) - 1)
    def _():
        o_ref[...] = jnp.maximum(o_ref[...] + b_ref[...], 0.0)


def _expert_ffn(e_arr, x, w_moe, b_moe, layer, mb, kb):
    m = x.shape[0]
    return pl.pallas_call(
        _ffn_kernel,
        grid_spec=pltpu.PrefetchScalarGridSpec(
            num_scalar_prefetch=1,
            grid=(m // mb, _HID // kb),
            in_specs=[
                pl.BlockSpec((mb, kb), lambda mi, k, e: (mi, k)),
                pl.BlockSpec((None, None, None, kb, _HID),
                             lambda mi, k, e: (layer, 7, e[0], k, 0)),
                pl.BlockSpec((None, 1, _HID),
                             lambda mi, k, e: (layer * 8 * _NE + 7 * _NE + e[0], 0, 0)),
            ],
            out_specs=pl.BlockSpec((mb, _HID), lambda mi, k, e: (mi, 0)),
        ),
        out_shape=jax.ShapeDtypeStruct((m, _HID), _F32),
        compiler_params=pltpu.CompilerParams(
            dimension_semantics=("parallel", "arbitrary")),
    )(e_arr, x, w_moe, b_moe)


def _attn_kernel(q_ref, kva_ref, kvt_ref, cq_ref, aq_ref, bq_ref,
                 ck_ref, ak_ref, bk_ref, o_ref, nb):
    cq, aq, bq = cq_ref[...], aq_ref[...], bq_ref[...]
    ck, ak, bk = ck_ref[...], ak_ref[...], bk_ref[...]
    for j in range(nb):
        q = q_ref[j * _T:(j + 1) * _T, :]
        qr = (q * cq + _roll_l(q) * aq + _roll_r(q) * bq).astype(_BF)
        ka = kva_ref[j * _K:(j + 1) * _K, :_HID]
        kar = (ka * ck + _roll_l(ka) * ak + _roll_r(ka) * bk).astype(_BF)
        kt = kvt_ref[j * _K:(j + 1) * _K, :_HID]
        ktr = (kt * ck + _roll_l(kt) * ak + _roll_r(kt) * bk).astype(_BF)
        va = kva_ref[j * _K:(j + 1) * _K, _HID:].astype(_BF)
        vt = kvt_ref[j * _K:(j + 1) * _K, _HID:].astype(_BF)
        dn = (((1,), (1,)), ((), ()))
        for h in range(_NH):
            sl = slice(h * _HD, (h + 1) * _HD)
            sa = jax.lax.dot_general(qr[:, sl], kar[:, sl], dn,
                                     preferred_element_type=_F32)
            st = jax.lax.dot_general(qr[:, _HID + h * _HD:_HID + (h + 1) * _HD],
                                     ktr[:, sl], dn,
                                     preferred_element_type=_F32)
            s = jnp.concatenate([sa, st], axis=1)
            mx = jnp.max(s, axis=1, keepdims=True)
            p = jnp.exp(s - mx)
            w = (p / jnp.sum(p, axis=1, keepdims=True)).astype(_BF)
            v = jnp.concatenate([va[:, sl], vt[:, sl]], axis=0)
            o_ref[j * _T:(j + 1) * _T, sl] = jnp.dot(
                w, v, preferred_element_type=_F32)


def _attention(qq, kva, kvt, cq, aq, bq, ck, ak, bk, nb):
    grid = _B // nb
    return pl.pallas_call(
        functools.partial(_attn_kernel, nb=nb),
        grid=(grid,),
        in_specs=[
            pl.BlockSpec((nb * _T, 2 * _HID), lambda i: (i, 0)),
            pl.BlockSpec((nb * _K, 2 * _HID), lambda i: (i, 0)),
            pl.BlockSpec((nb * _K, 2 * _HID), lambda i: (i, 0)),
            pl.BlockSpec((_T, 2 * _HID), lambda i: (0, 0)),
            pl.BlockSpec((_T, 2 * _HID), lambda i: (0, 0)),
            pl.BlockSpec((_T, 2 * _HID), lambda i: (0, 0)),
            pl.BlockSpec((_K, _HID), lambda i: (0, 0)),
            pl.BlockSpec((_K, _HID), lambda i: (0, 0)),
            pl.BlockSpec((_K, _HID), lambda i: (0, 0)),
        ],
        out_specs=pl.BlockSpec((nb * _T, _HID), lambda i: (i, 0)),
        out_shape=jax.ShapeDtypeStruct((_B * _T, _HID), _F32),
        compiler_params=pltpu.CompilerParams(
            dimension_semantics=("parallel",)),
    )(qq, kva, kvt, cq, aq, bq, ck, ak, bk)


def kernel(x, h_a, h_t, W_moe, b_moe, norm_g, norm_b, gate, W_in, b_in,
           W_out, b_out, expert_idx):
    e_arr = jnp.asarray(expert_idx, dtype=jnp.int32).reshape((1,))

    xf = x.reshape(_B * _T, _IN_DIM)
    haf = h_a.reshape(_B * _K, _HID)
    htf = h_t.reshape(_B * _K, _HID)
    bm3 = b_moe.reshape(_NB * 8 * _NE, 1, _HID)
    ng3 = norm_g.reshape(_NB * _NE, 1, _HID)
    nb3 = norm_b.reshape(_NB * _NE, 1, _HID)

    cq0, aq0, bq0 = _rope_coeff(_T, _HD, 2 * _NH)
    ck0, ak0, bk0 = _rope_coeff(_K, _HD, _NH)
    inv = 1.0 / math.sqrt(_HD)

    h = _matmul_bias(xf, W_in, b_in.reshape(1, _HID), mb=512, kb=512)

    for layer in range(_NB):
        ratio = jax.nn.sigmoid(gate[layer, e_arr[0]])
        scale = jnp.concatenate(
            [jnp.full((1, _HID), inv, _F32),
             jnp.full((1, _HID), inv, _F32) * ratio], axis=1)
        cq, aq, bq = cq0 * scale, aq0 * scale, bq0 * scale

        qq = _expert_dual(e_arr, h, W_moe, bm3, layer, 0, 3, mb=512, kb=512)
        kva = _expert_dual(e_arr, haf, W_moe, bm3, layer, 1, 2, mb=512, kb=512)
        kvt = _expert_dual(e_arr, htf, W_moe, bm3, layer, 4, 5, mb=512, kb=512)
        attn = _attention(qq, kva, kvt, cq, aq, bq, ck0, ak0, bk0, nb=8)
        hn = _expert_oln(e_arr, attn, W_moe, bm3, h, ng3, nb3,
                         layer, mb=512, kb=512)
        h = _expert_ffn(e_arr, hn, W_moe, bm3, layer, mb=512, kb=512)

    w_out_p = jnp.zeros((_HID, 128), _F32).at[:, :_ACT].set(W_out)
    b_out_p = jnp.zeros((1, 128), _F32).at[:, :_ACT].set(b_out.reshape(1, _ACT))
    out = _matmul_bias(h, w_out_p, b_out_p, mb=512, kb=1024)
    return out[:, :_ACT].reshape(_B, _T, _ACT)


# speedup vs baseline: 2.2569x; 1.4155x over previous
import functools
import math

import jax
import jax.numpy as jnp
import numpy as np
from jax.experimental import pallas as pl
from jax.experimental.pallas import tpu as pltpu

_B, _T, _K = 64, 64, 32
_IN_DIM, _HID, _ACT = 4096, 1024, 7
_NE, _NH, _NB = 8, 8, 2
_HD = _HID // _NH

_BF = jnp.bfloat16
_F32 = jnp.float32


def _rope_tables(seq_len, dim):
    inv_freq = 1.0 / (10000.0 ** (jnp.arange(0, dim, 2, dtype=_F32) / dim))
    t = jnp.arange(seq_len, dtype=_F32)
    freqs = t[:, None] * inv_freq[None, :]
    emb = jnp.concatenate([freqs, freqs], axis=-1)
    return jnp.cos(emb), jnp.sin(emb)


def _rope_coeff(seq_len, dim):
    cos, sin = _rope_tables(seq_len, dim)
    lane = jnp.arange(dim)
    even = (lane % 2) == 0
    a = jnp.where(even[None, :], -sin, 0.0)
    b = jnp.where(even[None, :], 0.0, sin)
    return cos, a, b


def _roll_l(z):
    return jnp.concatenate([z[:, 1:], z[:, :1]], axis=1)


def _roll_r(z):
    return jnp.concatenate([z[:, -1:], z[:, :-1]], axis=1)


def _mm_kernel(x_ref, w_ref, b_ref, o_ref):
    k = pl.program_id(1)

    @pl.when(k == 0)
    def _():
        o_ref[...] = jnp.zeros_like(o_ref)

    o_ref[...] += jnp.dot(x_ref[...], w_ref[...], preferred_element_type=_F32)

    @pl.when(k == pl.num_programs(1) - 1)
    def _():
        o_ref[...] += b_ref[...]


def _matmul_bias(x, w, b_row, mb, kb):
    m, kk = x.shape
    n = w.shape[1]
    return pl.pallas_call(
        _mm_kernel,
        grid=(m // mb, kk // kb),
        in_specs=[
            pl.BlockSpec((mb, kb), lambda i, k: (i, k)),
            pl.BlockSpec((kb, n), lambda i, k: (k, 0)),
            pl.BlockSpec((1, n), lambda i, k: (0, 0)),
        ],
        out_specs=pl.BlockSpec((mb, n), lambda i, k: (i, 0)),
        out_shape=jax.ShapeDtypeStruct((m, n), _F32),
        compiler_params=pltpu.CompilerParams(
            dimension_semantics=("parallel", "arbitrary")),
    )(x, w, b_row)


def _dual_kernel(e_ref, x_ref, w1_ref, w2_ref, b1_ref, b2_ref, o_ref):
    xb = x_ref[...].astype(_BF)
    o_ref[:, :_HID] = (jnp.dot(xb, w1_ref[...].astype(_BF),
                               preferred_element_type=_F32)
                       + b1_ref[...]).astype(_BF)
    o_ref[:, _HID:] = (jnp.dot(xb, w2_ref[...].astype(_BF),
                               preferred_element_type=_F32)
                       + b2_ref[...]).astype(_BF)


def _expert_dual(e_arr, x, w_moe, b_moe, layer, i1, i2, mb):
    m = x.shape[0]
    wspec = lambda i: pl.BlockSpec(
        (None, None, None, _HID, _HID),
        lambda mi, e: (layer, i, e[0], 0, 0))
    bspec = lambda i: pl.BlockSpec(
        (None, 1, _HID), lambda mi, e: (layer * 8 * _NE + i * _NE + e[0], 0, 0))
    return pl.pallas_call(
        _dual_kernel,
        grid_spec=pltpu.PrefetchScalarGridSpec(
            num_scalar_prefetch=1,
            grid=(m // mb,),
            in_specs=[
                pl.BlockSpec((mb, _HID), lambda mi, e: (mi, 0)),
                wspec(i1), wspec(i2), bspec(i1), bspec(i2),
            ],
            out_specs=pl.BlockSpec((mb, 2 * _HID), lambda mi, e: (mi, 0)),
        ),
        out_shape=jax.ShapeDtypeStruct((m, 2 * _HID), _BF),
        compiler_params=pltpu.CompilerParams(
            dimension_semantics=("parallel",)),
    )(e_arr, x, w_moe, w_moe, b_moe, b_moe)


def _oln_kernel(e_ref, a_ref, w_ref, b_ref, res_ref, g_ref, be_ref, o_ref):
    y = jnp.dot(a_ref[...], w_ref[...].astype(_BF),
                preferred_element_type=_F32) + b_ref[...] + res_ref[...]
    mu = jnp.mean(y, axis=1, keepdims=True)
    d = y - mu
    var = jnp.mean(d * d, axis=1, keepdims=True)
    o_ref[...] = (d * jax.lax.rsqrt(var + 1e-5) * g_ref[...]
                  + be_ref[...]).astype(_BF)


def _expert_oln(e_arr, attn, w_moe, b_moe, res, norm_g, norm_b, layer, mb):
    m = attn.shape[0]
    return pl.pallas_call(
        _oln_kernel,
        grid_spec=pltpu.PrefetchScalarGridSpec(
            num_scalar_prefetch=1,
            grid=(m // mb,),
            in_specs=[
                pl.BlockSpec((mb, _HID), lambda mi, e: (mi, 0)),
                pl.BlockSpec((None, None, None, _HID, _HID),
                             lambda mi, e: (layer, 6, e[0], 0, 0)),
                pl.BlockSpec((None, 1, _HID),
                             lambda mi, e: (layer * 8 * _NE + 6 * _NE + e[0], 0, 0)),
                pl.BlockSpec((mb, _HID), lambda mi, e: (mi, 0)),
                pl.BlockSpec((None, 1, _HID),
                             lambda mi, e: (layer * _NE + e[0], 0, 0)),
                pl.BlockSpec((None, 1, _HID),
                             lambda mi, e: (layer * _NE + e[0], 0, 0)),
            ],
            out_specs=pl.BlockSpec((mb, _HID), lambda mi, e: (mi, 0)),
        ),
        out_shape=jax.ShapeDtypeStruct((m, _HID), _BF),
        compiler_params=pltpu.CompilerParams(
            dimension_semantics=("parallel",)),
    )(e_arr, attn, w_moe, b_moe, res, norm_g, norm_b)


def _ffn_kernel(e_ref, x_ref, w_ref, b_ref, o_ref):
    o_ref[...] = jnp.maximum(
        jnp.dot(x_ref[...], w_ref[...].astype(_BF),
                preferred_element_type=_F32) + b_ref[...], 0.0)


def _expert_ffn(e_arr, x, w_moe, b_moe, layer, mb):
    m = x.shape[0]
    return pl.pallas_call(
        _ffn_kernel,
        grid_spec=pltpu.PrefetchScalarGridSpec(
            num_scalar_prefetch=1,
            grid=(m // mb,),
            in_specs=[
                pl.BlockSpec((mb, _HID), lambda mi, e: (mi, 0)),
                pl.BlockSpec((None, None, None, _HID, _HID),
                             lambda mi, e: (layer, 7, e[0], 0, 0)),
                pl.BlockSpec((None, 1, _HID),
                             lambda mi, e: (layer * 8 * _NE + 7 * _NE + e[0], 0, 0)),
            ],
            out_specs=pl.BlockSpec((mb, _HID), lambda mi, e: (mi, 0)),
        ),
        out_shape=jax.ShapeDtypeStruct((m, _HID), _F32),
        compiler_params=pltpu.CompilerParams(
            dimension_semantics=("parallel",)),
    )(e_arr, x, w_moe, b_moe)


def _ffn_head_kernel(e_ref, x_ref, w_ref, b_ref, wo_ref, bo_ref, o_ref):
    t = jnp.maximum(
        jnp.dot(x_ref[...], w_ref[...].astype(_BF),
                preferred_element_type=_F32) + b_ref[...], 0.0)
    o_ref[...] = jnp.dot(t.astype(_BF), wo_ref[...],
                         preferred_element_type=_F32) + bo_ref[...]


def _expert_ffn_head(e_arr, x, w_moe, b_moe, w_out, b_out, layer, mb):
    m = x.shape[0]
    return pl.pallas_call(
        _ffn_head_kernel,
        grid_spec=pltpu.PrefetchScalarGridSpec(
            num_scalar_prefetch=1,
            grid=(m // mb,),
            in_specs=[
                pl.BlockSpec((mb, _HID), lambda mi, e: (mi, 0)),
                pl.BlockSpec((None, None, None, _HID, _HID),
                             lambda mi, e: (layer, 7, e[0], 0, 0)),
                pl.BlockSpec((None, 1, _HID),
                             lambda mi, e: (layer * 8 * _NE + 7 * _NE + e[0], 0, 0)),
                pl.BlockSpec((_HID, 128), lambda mi, e: (0, 0)),
                pl.BlockSpec((1, 128), lambda mi, e: (0, 0)),
            ],
            out_specs=pl.BlockSpec((mb, 128), lambda mi, e: (mi, 0)),
        ),
        out_shape=jax.ShapeDtypeStruct((m, 128), _F32),
        compiler_params=pltpu.CompilerParams(
            dimension_semantics=("parallel",)),
    )(e_arr, x, w_moe, b_moe, w_out, b_out)


def _attn_kernel(q_ref, kva_ref, kvt_ref, cqa_ref, aqa_ref, bqa_ref,
                 cqt_ref, aqt_ref, bqt_ref, ck_ref, ak_ref, bk_ref,
                 o_ref, nb):
    cqa, aqa, bqa = cqa_ref[...], aqa_ref[...], bqa_ref[...]
    cqt, aqt, bqt = cqt_ref[...], aqt_ref[...], bqt_ref[...]
    ck, ak, bk = ck_ref[...], ak_ref[...], bk_ref[...]
    dn = (((1,), (1,)), ((), ()))
    for j in range(nb):
        rq = slice(j * _T, (j + 1) * _T)
        rk = slice(j * _K, (j + 1) * _K)
        for h in range(_NH):
            sl = slice(h * _HD, (h + 1) * _HD)
            slt = slice(_HID + h * _HD, _HID + (h + 1) * _HD)
            qa = q_ref[rq, sl].astype(_F32)
            qar = (qa * cqa + _roll_l(qa) * aqa + _roll_r(qa) * bqa).astype(_BF)
            qt = q_ref[rq, slt].astype(_F32)
            qtr = (qt * cqt + _roll_l(qt) * aqt + _roll_r(qt) * bqt).astype(_BF)
            ka = kva_ref[rk, sl].astype(_F32)
            kar = (ka * ck + _roll_l(ka) * ak + _roll_r(ka) * bk).astype(_BF)
            kt = kvt_ref[rk, sl].astype(_F32)
            ktr = (kt * ck + _roll_l(kt) * ak + _roll_r(kt) * bk).astype(_BF)
            sa = jax.lax.dot_general(qar, kar, dn, preferred_element_type=_F32)
            st = jax.lax.dot_general(qtr, ktr, dn, preferred_element_type=_F32)
            mx = jnp.maximum(jnp.max(sa, axis=1, keepdims=True),
                             jnp.max(st, axis=1, keepdims=True))
            pa = jnp.exp(sa - mx)
            pt = jnp.exp(st - mx)
            den = (jnp.sum(pa, axis=1, keepdims=True)
                   + jnp.sum(pt, axis=1, keepdims=True))
            va = kva_ref[rk, slt]
            vt = kvt_ref[rk, slt]
            o = (jnp.dot(pa.astype(_BF), va, preferred_element_type=_F32)
                 + jnp.dot(pt.astype(_BF), vt, preferred_element_type=_F32))
            o_ref[rq, sl] = (o * (1.0 / den)).astype(_BF)


def _attention(qq, kva, kvt, qtab_a, qtab_t, ktab, nb):
    grid = _B // nb
    tspec = lambda r: pl.BlockSpec((r, _HD), lambda i: (0, 0))
    return pl.pallas_call(
        functools.partial(_attn_kernel, nb=nb),
        grid=(grid,),
        in_specs=[
            pl.BlockSpec((nb * _T, 2 * _HID), lambda i: (i, 0)),
            pl.BlockSpec((nb * _K, 2 * _HID), lambda i: (i, 0)),
            pl.BlockSpec((nb * _K, 2 * _HID), lambda i: (i, 0)),
            tspec(_T), tspec(_T), tspec(_T),
            tspec(_T), tspec(_T), tspec(_T),
            tspec(_K), tspec(_K), tspec(_K),
        ],
        out_specs=pl.BlockSpec((nb * _T, _HID), lambda i: (i, 0)),
        out_shape=jax.ShapeDtypeStruct((_B * _T, _HID), _BF),
        compiler_params=pltpu.CompilerParams(
            dimension_semantics=("parallel",)),
    )(qq, kva, kvt, *qtab_a, *qtab_t, *ktab)


def kernel(x, h_a, h_t, W_moe, b_moe, norm_g, norm_b, gate, W_in, b_in,
           W_out, b_out, expert_idx):
    e_arr = jnp.asarray(expert_idx, dtype=jnp.int32).reshape((1,))

    xf = x.reshape(_B * _T, _IN_DIM).astype(_BF)
    haf = h_a.reshape(_B * _K, _HID)
    htf = h_t.reshape(_B * _K, _HID)
    bm3 = b_moe.reshape(_NB * 8 * _NE, 1, _HID)
    ng3 = norm_g.reshape(_NB * _NE, 1, _HID)
    nb3 = norm_b.reshape(_NB * _NE, 1, _HID)

    cq0, aq0, bq0 = _rope_coeff(_T, _HD)
    ck0, ak0, bk0 = _rope_coeff(_K, _HD)
    inv = 1.0 / math.sqrt(_HD)
    qtab_a = (cq0 * inv, aq0 * inv, bq0 * inv)
    ktab = (ck0, ak0, bk0)

    h = _matmul_bias(xf, W_in.astype(_BF), b_in.reshape(1, _HID),
                     mb=512, kb=1024)

    for layer in range(_NB):
        s_t = inv * jax.nn.sigmoid(gate[layer, e_arr[0]])
        qtab_t = (cq0 * s_t, aq0 * s_t, bq0 * s_t)

        qq = _expert_dual(e_arr, h, W_moe, bm3, layer, 0, 3, mb=512)
        kva = _expert_dual(e_arr, haf, W_moe, bm3, layer, 1, 2, mb=512)
        kvt = _expert_dual(e_arr, htf, W_moe, bm3, layer, 4, 5, mb=512)
        attn = _attention(qq, kva, kvt, qtab_a, qtab_t, ktab, nb=4)
        hn = _expert_oln(e_arr, attn, W_moe, bm3, h, ng3, nb3, layer, mb=512)
        if layer < _NB - 1:
            h = _expert_ffn(e_arr, hn, W_moe, bm3, layer, mb=512)

    w_out_p = jnp.zeros((_HID, 128), _F32).at[:, :_ACT].set(W_out).astype(_BF)
    b_out_p = jnp.zeros((1, 128), _F32).at[:, :_ACT].set(b_out.reshape(1, _ACT))
    out = _expert_ffn_head(e_arr, hn, W_moe, bm3, w_out_p, b_out_p,
                           _NB - 1, mb=512)
    return out[:, :_ACT].reshape(_B, _T, _ACT)


# speedup vs baseline: 2.6560x; 1.1769x over previous
import functools
import math

import jax
import jax.numpy as jnp
import numpy as np
from jax.experimental import pallas as pl
from jax.experimental.pallas import tpu as pltpu

_B, _T, _K = 64, 64, 32
_IN_DIM, _HID, _ACT = 4096, 1024, 7
_NE, _NH, _NB = 8, 8, 2
_HD = _HID // _NH

_BF = jnp.bfloat16
_F32 = jnp.float32


def _np_rope_coeff(seq_len, rows, scale):
    inv_freq = 1.0 / (10000.0 ** (np.arange(0, _HD, 2, dtype=np.float64) / _HD))
    t = np.arange(seq_len, dtype=np.float64)
    freqs = t[:, None] * inv_freq[None, :]
    emb = np.concatenate([freqs, freqs], axis=-1)
    cos, sin = np.cos(emb), np.sin(emb)
    even = (np.arange(_HD) % 2) == 0
    a = np.where(even[None, :], -sin, 0.0)
    b = np.where(even[None, :], 0.0, sin)
    tile = lambda z: jnp.asarray(
        np.tile(z * scale, (rows // seq_len, _HID // _HD)).astype(np.float32))
    return tile(cos), tile(a), tile(b)


def _roll_l(z):
    return jnp.concatenate([z[:, 1:], z[:, :1]], axis=1)


def _roll_r(z):
    return jnp.concatenate([z[:, -1:], z[:, :-1]], axis=1)


def _rope(y, c, a, b):
    return y * c + _roll_l(y) * a + _roll_r(y) * b


def _mm_kernel(x_ref, w_ref, b_ref, o_ref):
    k = pl.program_id(1)

    @pl.when(k == 0)
    def _():
        o_ref[...] = jnp.zeros_like(o_ref)

    o_ref[...] += jnp.dot(x_ref[...], w_ref[...], preferred_element_type=_F32)

    @pl.when(k == pl.num_programs(1) - 1)
    def _():
        o_ref[...] += b_ref[...]


def _matmul_bias(x, w, b_row, mb, kb):
    m, kk = x.shape
    n = w.shape[1]
    return pl.pallas_call(
        _mm_kernel,
        grid=(m // mb, kk // kb),
        in_specs=[
            pl.BlockSpec((mb, kb), lambda i, k: (i, k)),
            pl.BlockSpec((kb, n), lambda i, k: (k, 0)),
            pl.BlockSpec((1, n), lambda i, k: (0, 0)),
        ],
        out_specs=pl.BlockSpec((mb, n), lambda i, k: (i, 0)),
        out_shape=jax.ShapeDtypeStruct((m, n), _F32),
        compiler_params=pltpu.CompilerParams(
            dimension_semantics=("parallel", "arbitrary")),
    )(x, w, b_row)


def _dual_rope_kernel(e_ref, x_ref, w1_ref, w2_ref, b1_ref, b2_ref,
                      c_ref, a_ref, b3_ref, o_ref, *, rope2):
    xb = x_ref[...].astype(_BF)
    c, a, b = c_ref[...], a_ref[...], b3_ref[...]
    acc1 = jnp.dot(xb, w1_ref[...].astype(_BF),
                   preferred_element_type=_F32) + b1_ref[...]
    o_ref[:, :_HID] = _rope(acc1, c, a, b).astype(_BF)
    acc2 = jnp.dot(xb, w2_ref[...].astype(_BF),
                   preferred_element_type=_F32) + b2_ref[...]
    if rope2:
        o_ref[:, _HID:] = _rope(acc2, c, a, b).astype(_BF)
    else:
        o_ref[:, _HID:] = acc2.astype(_BF)


def _qq_proj(e_arr, x, w_moe, b_moe, qtab, layer, mb):
    m = x.shape[0]
    wspec = lambda i: pl.BlockSpec(
        (None, None, None, _HID, _HID), lambda mi, e: (layer, i, e[0], 0, 0))
    bspec = lambda i: pl.BlockSpec(
        (None, 1, _HID), lambda mi, e: (layer * 8 * _NE + i * _NE + e[0], 0, 0))
    tspec = pl.BlockSpec((mb, _HID), lambda mi, e: (0, 0))
    return pl.pallas_call(
        functools.partial(_dual_rope_kernel, rope2=True),
        grid_spec=pltpu.PrefetchScalarGridSpec(
            num_scalar_prefetch=1,
            grid=(m // mb,),
            in_specs=[
                pl.BlockSpec((mb, _HID), lambda mi, e: (mi, 0)),
                wspec(0), wspec(3), bspec(0), bspec(3),
                tspec, tspec, tspec,
            ],
            out_specs=pl.BlockSpec((mb, 2 * _HID), lambda mi, e: (mi, 0)),
        ),
        out_shape=jax.ShapeDtypeStruct((m, 2 * _HID), _BF),
        compiler_params=pltpu.CompilerParams(
            dimension_semantics=("parallel",)),
    )(e_arr, x, w_moe, w_moe, b_moe, b_moe, *qtab)


def _kv_proj(e_arr, kv_in, w_moe, b_moe, ktab, mb):
    wspec = lambda which: pl.BlockSpec(
        (None, None, None, _HID, _HID),
        lambda mi, e: ((mi // 4) % 2, (mi // 8) * 3 + which, e[0], 0, 0))
    bspec = lambda which: pl.BlockSpec(
        (None, 1, _HID),
        lambda mi, e: (((mi // 4) % 2) * 8 * _NE
                       + ((mi // 8) * 3 + which) * _NE + e[0], 0, 0))
    tspec = pl.BlockSpec((mb, _HID), lambda mi, e: (0, 0))
    return pl.pallas_call(
        functools.partial(_dual_rope_kernel, rope2=False),
        grid_spec=pltpu.PrefetchScalarGridSpec(
            num_scalar_prefetch=1,
            grid=(16,),
            in_specs=[
                pl.BlockSpec((mb, _HID),
                             lambda mi, e: ((mi // 8) * 4 + mi % 4, 0)),
                wspec(1), wspec(2), bspec(1), bspec(2),
                tspec, tspec, tspec,
            ],
            out_specs=pl.BlockSpec((mb, 2 * _HID), lambda mi, e: (mi, 0)),
        ),
        out_shape=jax.ShapeDtypeStruct((16 * mb, 2 * _HID), _BF),
        compiler_params=pltpu.CompilerParams(
            dimension_semantics=("arbitrary",)),
    )(e_arr, kv_in, w_moe, w_moe, b_moe, b_moe, *ktab)


def _oln_kernel(e_ref, a_ref, w_ref, b_ref, res_ref, g_ref, be_ref, o_ref):
    y = jnp.dot(a_ref[...], w_ref[...].astype(_BF),
                preferred_element_type=_F32) + b_ref[...] + res_ref[...]
    mu = jnp.mean(y, axis=1, keepdims=True)
    d = y - mu
    var = jnp.mean(d * d, axis=1, keepdims=True)
    o_ref[...] = (d * jax.lax.rsqrt(var + 1e-5) * g_ref[...]
                  + be_ref[...]).astype(_BF)


def _expert_oln(e_arr, attn, w_moe, b_moe, res, norm_g, norm_b, layer, mb):
    m = attn.shape[0]
    return pl.pallas_call(
        _oln_kernel,
        grid_spec=pltpu.PrefetchScalarGridSpec(
            num_scalar_prefetch=1,
            grid=(m // mb,),
            in_specs=[
                pl.BlockSpec((mb, _HID), lambda mi, e: (mi, 0)),
                pl.BlockSpec((None, None, None, _HID, _HID),
                             lambda mi, e: (layer, 6, e[0], 0, 0)),
                pl.BlockSpec((None, 1, _HID),
                             lambda mi, e: (layer * 8 * _NE + 6 * _NE + e[0], 0, 0)),
                pl.BlockSpec((mb, _HID), lambda mi, e: (mi, 0)),
                pl.BlockSpec((None, 1, _HID),
                             lambda mi, e: (layer * _NE + e[0], 0, 0)),
                pl.BlockSpec((None, 1, _HID),
                             lambda mi, e: (layer * _NE + e[0], 0, 0)),
            ],
            out_specs=pl.BlockSpec((mb, _HID), lambda mi, e: (mi, 0)),
        ),
        out_shape=jax.ShapeDtypeStruct((m, _HID), _BF),
        compiler_params=pltpu.CompilerParams(
            dimension_semantics=("parallel",)),
    )(e_arr, attn, w_moe, b_moe, res, norm_g, norm_b)


def _ffn_kernel(e_ref, x_ref, w_ref, b_ref, o_ref):
    o_ref[...] = jnp.maximum(
        jnp.dot(x_ref[...], w_ref[...].astype(_BF),
                preferred_element_type=_F32) + b_ref[...], 0.0)


def _expert_ffn(e_arr, x, w_moe, b_moe, layer, mb):
    m = x.shape[0]
    return pl.pallas_call(
        _ffn_kernel,
        grid_spec=pltpu.PrefetchScalarGridSpec(
            num_scalar_prefetch=1,
            grid=(m // mb,),
            in_specs=[
                pl.BlockSpec((mb, _HID), lambda mi, e: (mi, 0)),
                pl.BlockSpec((None, None, None, _HID, _HID),
                             lambda mi, e: (layer, 7, e[0], 0, 0)),
                pl.BlockSpec((None, 1, _HID),
                             lambda mi, e: (layer * 8 * _NE + 7 * _NE + e[0], 0, 0)),
            ],
            out_specs=pl.BlockSpec((mb, _HID), lambda mi, e: (mi, 0)),
        ),
        out_shape=jax.ShapeDtypeStruct((m, _HID), _F32),
        compiler_params=pltpu.CompilerParams(
            dimension_semantics=("parallel",)),
    )(e_arr, x, w_moe, b_moe)


def _ffn_head_kernel(e_ref, x_ref, w_ref, b_ref, wo_ref, bo_ref, o_ref):
    t = jnp.maximum(
        jnp.dot(x_ref[...], w_ref[...].astype(_BF),
                preferred_element_type=_F32) + b_ref[...], 0.0)
    o_ref[...] = jnp.dot(t.astype(_BF), wo_ref[...],
                         preferred_element_type=_F32) + bo_ref[...]


def _expert_ffn_head(e_arr, x, w_moe, b_moe, w_out, b_out, layer, mb):
    m = x.shape[0]
    return pl.pallas_call(
        _ffn_head_kernel,
        grid_spec=pltpu.PrefetchScalarGridSpec(
            num_scalar_prefetch=1,
            grid=(m // mb,),
            in_specs=[
                pl.BlockSpec((mb, _HID), lambda mi, e: (mi, 0)),
                pl.BlockSpec((None, None, None, _HID, _HID),
                             lambda mi, e: (layer, 7, e[0], 0, 0)),
                pl.BlockSpec((None, 1, _HID),
                             lambda mi, e: (layer * 8 * _NE + 7 * _NE + e[0], 0, 0)),
                pl.BlockSpec((_HID, 128), lambda mi, e: (0, 0)),
                pl.BlockSpec((1, 128), lambda mi, e: (0, 0)),
            ],
            out_specs=pl.BlockSpec((mb, 128), lambda mi, e: (mi, 0)),
        ),
        out_shape=jax.ShapeDtypeStruct((m, 128), _F32),
        compiler_params=pltpu.CompilerParams(
            dimension_semantics=("parallel",)),
    )(e_arr, x, w_moe, b_moe, w_out, b_out)


def _attn_kernel(q_ref, kva_ref, kvt_ref, rv_ref, o_ref, nb):
    rv = rv_ref[...]
    zero = jnp.zeros((_K, _HD), _BF)
    dn = (((1,), (1,)), ((), ()))
    for j in range(nb):
        rq = slice(j * _T, (j + 1) * _T)
        rk = slice(j * _K, (j + 1) * _K)
        for h in range(_NH):
            sl = slice(h * _HD, (h + 1) * _HD)
            slt = slice(_HID + h * _HD, _HID + (h + 1) * _HD)
            q2 = jnp.concatenate([q_ref[rq, sl], q_ref[rq, slt]], axis=1)
            k2 = jnp.concatenate(
                [jnp.concatenate([kva_ref[rk, sl], zero], axis=1),
                 jnp.concatenate([zero, kvt_ref[rk, sl]], axis=1)], axis=0)
            s = jax.lax.dot_general(q2, k2, dn,
                                    preferred_element_type=_F32) * rv
            mx = jnp.max(s, axis=1, keepdims=True)
            p = jnp.exp(s - mx)
            den = jnp.sum(p, axis=1, keepdims=True)
            w = (p * (1.0 / den)).astype(_BF)
            v2 = jnp.concatenate([kva_ref[rk, slt], kvt_ref[rk, slt]], axis=0)
            o_ref[rq, sl] = jnp.dot(w, v2,
                                    preferred_element_type=_F32).astype(_BF)


def _attention(qq, kv, rvec, layer, nb):
    grid = _B // nb
    kb = (nb * _K) // 128
    return pl.pallas_call(
        functools.partial(_attn_kernel, nb=nb),
        grid=(grid,),
        in_specs=[
            pl.BlockSpec((nb * _T, 2 * _HID), lambda i: (i, 0)),
            pl.BlockSpec((nb * _K, 2 * _HID),
                         lambda i: (layer * (16 // kb) + i, 0)),
            pl.BlockSpec((nb * _K, 2 * _HID),
                         lambda i: ((32 + layer * 16) // kb + i, 0)),
            pl.BlockSpec((_T, _T), lambda i: (0, 0)),
        ],
        out_specs=pl.BlockSpec((nb * _T, _HID), lambda i: (i, 0)),
        out_shape=jax.ShapeDtypeStruct((_B * _T, _HID), _BF),
        compiler_params=pltpu.CompilerParams(
            dimension_semantics=("parallel",)),
    )(qq, kv, kv, rvec)


def kernel(x, h_a, h_t, W_moe, b_moe, norm_g, norm_b, gate, W_in, b_in,
           W_out, b_out, expert_idx):
    e_arr = jnp.asarray(expert_idx, dtype=jnp.int32).reshape((1,))

    xf = x.reshape(_B * _T, _IN_DIM).astype(_BF)
    kv_in = jnp.concatenate([h_a.reshape(_B * _K, _HID),
                             h_t.reshape(_B * _K, _HID)], axis=0)
    bm3 = b_moe.reshape(_NB * 8 * _NE, 1, _HID)
    ng3 = norm_g.reshape(_NB * _NE, 1, _HID)
    nb3 = norm_b.reshape(_NB * _NE, 1, _HID)

    inv = 1.0 / math.sqrt(_HD)
    qtab = _np_rope_coeff(_T, 512, inv)
    ktab = _np_rope_coeff(_K, 512, 1.0)

    h = _matmul_bias(xf, W_in.astype(_BF), b_in.reshape(1, _HID),
                     mb=512, kb=1024)
    kv = _kv_proj(e_arr, kv_in, W_moe, bm3, ktab, mb=512)

    lane64 = jnp.arange(_T)[None, :] < _K
    for layer in range(_NB):
        ratio = jax.nn.sigmoid(gate[layer, e_arr[0]])
        rvec = jnp.broadcast_to(jnp.where(lane64, 1.0, ratio), (_T, _T))

        qq = _qq_proj(e_arr, h, W_moe, bm3, qtab, layer, mb=512)
        attn = _attention(qq, kv, rvec, layer, nb=4)
        hn = _expert_oln(e_arr, attn, W_moe, bm3, h, ng3, nb3, layer, mb=512)
        if layer < _NB - 1:
            h = _expert_ffn(e_arr, hn, W_moe, bm3, layer, mb=512)

    w_out_p = jnp.zeros((_HID, 128), _F32).at[:, :_ACT].set(W_out).astype(_BF)
    b_out_p = jnp.zeros((1, 128), _F32).at[:, :_ACT].set(b_out.reshape(1, _ACT))
    out = _expert_ffn_head(e_arr, hn, W_moe, bm3, w_out_p, b_out_p,
                           _NB - 1, mb=512)
    return out[:, :_ACT].reshape(_B, _T, _ACT)


# speedup vs baseline: 3.4961x; 1.3163x over previous
import functools
import math

import jax
import jax.numpy as jnp
import numpy as np
from jax.experimental import pallas as pl
from jax.experimental.pallas import tpu as pltpu

_B, _T, _K = 64, 64, 32
_IN_DIM, _HID, _ACT = 4096, 1024, 7
_NE, _NH, _NB = 8, 8, 2
_HD = _HID // _NH

_BF = jnp.bfloat16
_F32 = jnp.float32


def _np_rope_coeff(seq_len, rows, scale):
    inv_freq = 1.0 / (10000.0 ** (np.arange(0, _HD, 2, dtype=np.float64) / _HD))
    t = np.arange(seq_len, dtype=np.float64)
    freqs = t[:, None] * inv_freq[None, :]
    emb = np.concatenate([freqs, freqs], axis=-1)
    cos, sin = np.cos(emb), np.sin(emb)
    even = (np.arange(_HD) % 2) == 0
    a = np.where(even[None, :], -sin, 0.0)
    b = np.where(even[None, :], 0.0, sin)
    tile = lambda z: jnp.asarray(
        np.tile(z * scale, (rows // seq_len, _HID // _HD)).astype(np.float32))
    return tile(cos), tile(a), tile(b)


def _roll_l(z):
    return jnp.concatenate([z[:, 1:], z[:, :1]], axis=1)


def _roll_r(z):
    return jnp.concatenate([z[:, -1:], z[:, :-1]], axis=1)


def _rope(y, c, a, b):
    return y * c + _roll_l(y) * a + _roll_r(y) * b


def _mm_kernel(x_ref, w_ref, b_ref, o_ref):
    k = pl.program_id(1)

    @pl.when(k == 0)
    def _():
        o_ref[...] = jnp.zeros_like(o_ref)

    o_ref[...] += jnp.dot(x_ref[...], w_ref[...], preferred_element_type=_F32)

    @pl.when(k == pl.num_programs(1) - 1)
    def _():
        o_ref[...] += b_ref[...]


def _matmul_bias(x, w, b_row, mb, kb):
    m, kk = x.shape
    n = w.shape[1]
    return pl.pallas_call(
        _mm_kernel,
        grid=(m // mb, kk // kb),
        in_specs=[
            pl.BlockSpec((mb, kb), lambda i, k: (i, k)),
            pl.BlockSpec((kb, n), lambda i, k: (k, 0)),
            pl.BlockSpec((1, n), lambda i, k: (0, 0)),
        ],
        out_specs=pl.BlockSpec((mb, n), lambda i, k: (i, 0)),
        out_shape=jax.ShapeDtypeStruct((m, n), _F32),
        compiler_params=pltpu.CompilerParams(
            dimension_semantics=("parallel", "arbitrary")),
    )(x, w, b_row)


def _dual_rope_kernel(e_ref, x_ref, w1_ref, w2_ref, b1_ref, b2_ref,
                      c_ref, a_ref, b3_ref, o_ref, *, rope2):
    xb = x_ref[...].astype(_BF)
    c, a, b = c_ref[...], a_ref[...], b3_ref[...]
    acc1 = jnp.dot(xb, w1_ref[...].astype(_BF),
                   preferred_element_type=_F32) + b1_ref[...]
    o_ref[:, :_HID] = _rope(acc1, c, a, b).astype(_BF)
    acc2 = jnp.dot(xb, w2_ref[...].astype(_BF),
                   preferred_element_type=_F32) + b2_ref[...]
    if rope2:
        o_ref[:, _HID:] = _rope(acc2, c, a, b).astype(_BF)
    else:
        o_ref[:, _HID:] = acc2.astype(_BF)


def _qq_proj(e_arr, x, w_moe, b_moe, qtab, layer, mb):
    m = x.shape[0]
    wspec = lambda i: pl.BlockSpec(
        (None, None, None, _HID, _HID), lambda mi, e: (layer, i, e[0], 0, 0))
    bspec = lambda i: pl.BlockSpec(
        (None, 1, _HID), lambda mi, e: (layer * 8 * _NE + i * _NE + e[0], 0, 0))
    tspec = pl.BlockSpec((mb, _HID), lambda mi, e: (0, 0))
    return pl.pallas_call(
        functools.partial(_dual_rope_kernel, rope2=True),
        grid_spec=pltpu.PrefetchScalarGridSpec(
            num_scalar_prefetch=1,
            grid=(m // mb,),
            in_specs=[
                pl.BlockSpec((mb, _HID), lambda mi, e: (mi, 0)),
                wspec(0), wspec(3), bspec(0), bspec(3),
                tspec, tspec, tspec,
            ],
            out_specs=pl.BlockSpec((mb, 2 * _HID), lambda mi, e: (mi, 0)),
        ),
        out_shape=jax.ShapeDtypeStruct((m, 2 * _HID), _BF),
        compiler_params=pltpu.CompilerParams(
            dimension_semantics=("parallel",)),
    )(e_arr, x, w_moe, w_moe, b_moe, b_moe, *qtab)


def _kv_proj(e_arr, kv_in, w_moe, b_moe, ktab, mb):
    wspec = lambda which: pl.BlockSpec(
        (None, None, None, _HID, _HID),
        lambda mi, e: ((mi // 4) % 2, (mi // 8) * 3 + which, e[0], 0, 0))
    bspec = lambda which: pl.BlockSpec(
        (None, 1, _HID),
        lambda mi, e: (((mi // 4) % 2) * 8 * _NE
                       + ((mi // 8) * 3 + which) * _NE + e[0], 0, 0))
    tspec = pl.BlockSpec(
        (None, mb, _HID),
        lambda mi, e: ((mi // 8) * (1 + (mi // 4) % 2), 0, 0))
    return pl.pallas_call(
        functools.partial(_dual_rope_kernel, rope2=False),
        grid_spec=pltpu.PrefetchScalarGridSpec(
            num_scalar_prefetch=1,
            grid=(16,),
            in_specs=[
                pl.BlockSpec((mb, _HID),
                             lambda mi, e: ((mi // 8) * 4 + mi % 4, 0)),
                wspec(1), wspec(2), bspec(1), bspec(2),
                tspec, tspec, tspec,
            ],
            out_specs=pl.BlockSpec((mb, 2 * _HID), lambda mi, e: (mi, 0)),
        ),
        out_shape=jax.ShapeDtypeStruct((16 * mb, 2 * _HID), _BF),
        compiler_params=pltpu.CompilerParams(
            dimension_semantics=("arbitrary",)),
    )(e_arr, kv_in, w_moe, w_moe, b_moe, b_moe, *ktab)


def _oln_kernel(e_ref, a_ref, w_ref, b_ref, res_ref, g_ref, be_ref, o_ref):
    y = jnp.dot(a_ref[...], w_ref[...].astype(_BF),
                preferred_element_type=_F32) + b_ref[...] + res_ref[...]
    mu = jnp.mean(y, axis=1, keepdims=True)
    d = y - mu
    var = jnp.mean(d * d, axis=1, keepdims=True)
    o_ref[...] = (d * jax.lax.rsqrt(var + 1e-5) * g_ref[...]
                  + be_ref[...]).astype(_BF)


def _expert_oln(e_arr, attn, w_moe, b_moe, res, norm_g, norm_b, layer, mb):
    m = attn.shape[0]
    return pl.pallas_call(
        _oln_kernel,
        grid_spec=pltpu.PrefetchScalarGridSpec(
            num_scalar_prefetch=1,
            grid=(m // mb,),
            in_specs=[
                pl.BlockSpec((mb, _HID), lambda mi, e: (mi, 0)),
                pl.BlockSpec((None, None, None, _HID, _HID),
                             lambda mi, e: (layer, 6, e[0], 0, 0)),
                pl.BlockSpec((None, 1, _HID),
                             lambda mi, e: (layer * 8 * _NE + 6 * _NE + e[0], 0, 0)),
                pl.BlockSpec((mb, _HID), lambda mi, e: (mi, 0)),
                pl.BlockSpec((None, 1, _HID),
                             lambda mi, e: (layer * _NE + e[0], 0, 0)),
                pl.BlockSpec((None, 1, _HID),
                             lambda mi, e: (layer * _NE + e[0], 0, 0)),
            ],
            out_specs=pl.BlockSpec((mb, _HID), lambda mi, e: (mi, 0)),
        ),
        out_shape=jax.ShapeDtypeStruct((m, _HID), _BF),
        compiler_params=pltpu.CompilerParams(
            dimension_semantics=("parallel",)),
    )(e_arr, attn, w_moe, b_moe, res, norm_g, norm_b)


def _ffn_kernel(e_ref, x_ref, w_ref, b_ref, o_ref):
    o_ref[...] = jnp.maximum(
        jnp.dot(x_ref[...], w_ref[...].astype(_BF),
                preferred_element_type=_F32) + b_ref[...], 0.0)


def _expert_ffn(e_arr, x, w_moe, b_moe, layer, mb):
    m = x.shape[0]
    return pl.pallas_call(
        _ffn_kernel,
        grid_spec=pltpu.PrefetchScalarGridSpec(
            num_scalar_prefetch=1,
            grid=(m // mb,),
            in_specs=[
                pl.BlockSpec((mb, _HID), lambda mi, e: (mi, 0)),
                pl.BlockSpec((None, None, None, _HID, _HID),
                             lambda mi, e: (layer, 7, e[0], 0, 0)),
                pl.BlockSpec((None, 1, _HID),
                             lambda mi, e: (layer * 8 * _NE + 7 * _NE + e[0], 0, 0)),
            ],
            out_specs=pl.BlockSpec((mb, _HID), lambda mi, e: (mi, 0)),
        ),
        out_shape=jax.ShapeDtypeStruct((m, _HID), _F32),
        compiler_params=pltpu.CompilerParams(
            dimension_semantics=("parallel",)),
    )(e_arr, x, w_moe, b_moe)


def _ffn_head_kernel(e_ref, x_ref, w_ref, b_ref, wo_ref, bo_ref, o_ref):
    t = jnp.maximum(
        jnp.dot(x_ref[...], w_ref[...].astype(_BF),
                preferred_element_type=_F32) + b_ref[...], 0.0)
    o_ref[...] = jnp.dot(t.astype(_BF), wo_ref[...],
                         preferred_element_type=_F32) + bo_ref[...]


def _expert_ffn_head(e_arr, x, w_moe, b_moe, w_out, b_out, layer, mb):
    m = x.shape[0]
    return pl.pallas_call(
        _ffn_head_kernel,
        grid_spec=pltpu.PrefetchScalarGridSpec(
            num_scalar_prefetch=1,
            grid=(m // mb,),
            in_specs=[
                pl.BlockSpec((mb, _HID), lambda mi, e: (mi, 0)),
                pl.BlockSpec((None, None, None, _HID, _HID),
                             lambda mi, e: (layer, 7, e[0], 0, 0)),
                pl.BlockSpec((None, 1, _HID),
                             lambda mi, e: (layer * 8 * _NE + 7 * _NE + e[0], 0, 0)),
                pl.BlockSpec((_HID, 128), lambda mi, e: (0, 0)),
                pl.BlockSpec((1, 128), lambda mi, e: (0, 0)),
            ],
            out_specs=pl.BlockSpec((mb, 128), lambda mi, e: (mi, 0)),
        ),
        out_shape=jax.ShapeDtypeStruct((m, 128), _F32),
        compiler_params=pltpu.CompilerParams(
            dimension_semantics=("parallel",)),
    )(e_arr, x, w_moe, b_moe, w_out, b_out)


def _attn_kernel(q_ref, kva_ref, kvt_ref, o_ref, nb):
    zero = jnp.zeros((_K, _HD), _BF)
    dn = (((1,), (1,)), ((), ()))
    for j in range(nb):
        rq = slice(j * _T, (j + 1) * _T)
        rk = slice(j * _K, (j + 1) * _K)
        ss = []
        for h in range(_NH):
            sl = slice(h * _HD, (h + 1) * _HD)
            slt = slice(_HID + h * _HD, _HID + (h + 1) * _HD)
            q2 = jnp.concatenate([q_ref[rq, sl], q_ref[rq, slt]], axis=1)
            k2 = jnp.concatenate(
                [jnp.concatenate([kva_ref[rk, sl], zero], axis=1),
                 jnp.concatenate([zero, kvt_ref[rk, sl]], axis=1)], axis=0)
            ss.append(jax.lax.dot_general(q2, k2, dn,
                                          preferred_element_type=_F32))
        for h in range(_NH):
            sl = slice(h * _HD, (h + 1) * _HD)
            slt = slice(_HID + h * _HD, _HID + (h + 1) * _HD)
            s = ss[h]
            mx = jnp.max(s, axis=1, keepdims=True)
            p = jnp.exp(s - mx)
            den = jnp.sum(p, axis=1, keepdims=True)
            w = (p * (1.0 / den)).astype(_BF)
            v2 = jnp.concatenate([kva_ref[rk, slt], kvt_ref[rk, slt]], axis=0)
            o_ref[rq, sl] = jnp.dot(w, v2,
                                    preferred_element_type=_F32).astype(_BF)


def _attention(qq, kv, layer, nb):
    grid = _B // nb
    kb = (nb * _K) // 128
    return pl.pallas_call(
        functools.partial(_attn_kernel, nb=nb),
        grid=(grid,),
        in_specs=[
            pl.BlockSpec((nb * _T, 2 * _HID), lambda i: (i, 0)),
            pl.BlockSpec((nb * _K, 2 * _HID),
                         lambda i: (layer * (16 // kb) + i, 0)),
            pl.BlockSpec((nb * _K, 2 * _HID),
                         lambda i: ((32 + layer * 16) // kb + i, 0)),
        ],
        out_specs=pl.BlockSpec((nb * _T, _HID), lambda i: (i, 0)),
        out_shape=jax.ShapeDtypeStruct((_B * _T, _HID), _BF),
        compiler_params=pltpu.CompilerParams(
            dimension_semantics=("parallel",)),
    )(qq, kv, kv)


def kernel(x, h_a, h_t, W_moe, b_moe, norm_g, norm_b, gate, W_in, b_in,
           W_out, b_out, expert_idx):
    e_arr = jnp.asarray(expert_idx, dtype=jnp.int32).reshape((1,))

    xf = x.reshape(_B * _T, _IN_DIM).astype(_BF)
    kv_in = jnp.concatenate([h_a.reshape(_B * _K, _HID),
                             h_t.reshape(_B * _K, _HID)], axis=0)
    bm3 = b_moe.reshape(_NB * 8 * _NE, 1, _HID)
    ng3 = norm_g.reshape(_NB * _NE, 1, _HID)
    nb3 = norm_b.reshape(_NB * _NE, 1, _HID)

    inv = 1.0 / math.sqrt(_HD)
    qtab = _np_rope_coeff(_T, 512, inv)
    kc, ka, kb_ = _np_rope_coeff(_K, 512, 1.0)
    r0 = jax.nn.sigmoid(gate[0, e_arr[0]])
    r1 = jax.nn.sigmoid(gate[1, e_arr[0]])
    ktab = tuple(jnp.stack([z, z * r0, z * r1]) for z in (kc, ka, kb_))

    h = _matmul_bias(xf, W_in.astype(_BF), b_in.reshape(1, _HID),
                     mb=2048, kb=1024)
    kv = _kv_proj(e_arr, kv_in, W_moe, bm3, ktab, mb=512)

    for layer in range(_NB):
        qq = _qq_proj(e_arr, h, W_moe, bm3, qtab, layer, mb=512)
        attn = _attention(qq, kv, layer, nb=8)
        hn = _expert_oln(e_arr, attn, W_moe, bm3, h, ng3, nb3, layer, mb=512)
        if layer < _NB - 1:
            h = _expert_ffn(e_arr, hn, W_moe, bm3, layer, mb=512)

    w_out_p = jnp.zeros((_HID, 128), _F32).at[:, :_ACT].set(W_out).astype(_BF)
    b_out_p = jnp.zeros((1, 128), _F32).at[:, :_ACT].set(b_out.reshape(1, _ACT))
    out = _expert_ffn_head(e_arr, hn, W_moe, bm3, w_out_p, b_out_p,
                           _NB - 1, mb=512)
    return out[:, :_ACT].reshape(_B, _T, _ACT)


# speedup vs baseline: 3.6035x; 1.0307x over previous
import functools
import math

import jax
import jax.numpy as jnp
import numpy as np
from jax.experimental import pallas as pl
from jax.experimental.pallas import tpu as pltpu

_B, _T, _K = 64, 64, 32
_IN_DIM, _HID, _ACT = 4096, 1024, 7
_NE, _NH, _NB = 8, 8, 2
_HD = _HID // _NH

_BF = jnp.bfloat16
_F32 = jnp.float32


def _np_rope_coeff(seq_len, rows, scale):
    inv_freq = 1.0 / (10000.0 ** (np.arange(0, _HD, 2, dtype=np.float64) / _HD))
    t = np.arange(seq_len, dtype=np.float64)
    freqs = t[:, None] * inv_freq[None, :]
    emb = np.concatenate([freqs, freqs], axis=-1)
    cos, sin = np.cos(emb), np.sin(emb)
    even = (np.arange(_HD) % 2) == 0
    a = np.where(even[None, :], -sin, 0.0)
    b = np.where(even[None, :], 0.0, sin)
    tile = lambda z: jnp.asarray(
        np.tile(z * scale, (rows // seq_len, _HID // _HD)).astype(np.float32))
    return tile(cos), tile(a), tile(b)


def _roll_l(z):
    return jnp.concatenate([z[:, 1:], z[:, :1]], axis=1)


def _roll_r(z):
    return jnp.concatenate([z[:, -1:], z[:, :-1]], axis=1)


def _rope(y, c, a, b):
    return y * c + _roll_l(y) * a + _roll_r(y) * b


def _mm_kernel(x_ref, w_ref, b_ref, o_ref):
    k = pl.program_id(1)

    @pl.when(k == 0)
    def _():
        o_ref[...] = jnp.zeros_like(o_ref)

    o_ref[...] += jnp.dot(x_ref[...], w_ref[...], preferred_element_type=_F32)

    @pl.when(k == pl.num_programs(1) - 1)
    def _():
        o_ref[...] += b_ref[...]


def _matmul_bias(x, w, b_row, mb, kb):
    m, kk = x.shape
    n = w.shape[1]
    return pl.pallas_call(
        _mm_kernel,
        grid=(m // mb, kk // kb),
        in_specs=[
            pl.BlockSpec((mb, kb), lambda i, k: (i, k)),
            pl.BlockSpec((kb, n), lambda i, k: (k, 0)),
            pl.BlockSpec((1, n), lambda i, k: (0, 0)),
        ],
        out_specs=pl.BlockSpec((mb, n), lambda i, k: (i, 0)),
        out_shape=jax.ShapeDtypeStruct((m, n), _F32),
        compiler_params=pltpu.CompilerParams(
            dimension_semantics=("parallel", "arbitrary")),
    )(x, w, b_row)


def _dual_rope_kernel(e_ref, x_ref, w1_ref, w2_ref, b1_ref, b2_ref,
                      c_ref, a_ref, b3_ref, o_ref, *, rope2):
    xb = x_ref[...].astype(_BF)
    c, a, b = c_ref[...], a_ref[...], b3_ref[...]
    acc1 = jnp.dot(xb, w1_ref[...].astype(_BF),
                   preferred_element_type=_F32) + b1_ref[...]
    o_ref[:, :_HID] = _rope(acc1, c, a, b).astype(_BF)
    acc2 = jnp.dot(xb, w2_ref[...].astype(_BF),
                   preferred_element_type=_F32) + b2_ref[...]
    if rope2:
        o_ref[:, _HID:] = _rope(acc2, c, a, b).astype(_BF)
    else:
        o_ref[:, _HID:] = acc2.astype(_BF)


def _qq_proj(e_arr, x, w_moe, b_moe, qtab, layer, mb):
    m = x.shape[0]
    wspec = lambda i: pl.BlockSpec(
        (None, None, None, _HID, _HID), lambda mi, e: (layer, i, e[0], 0, 0))
    bspec = lambda i: pl.BlockSpec(
        (None, 1, _HID), lambda mi, e: (layer * 8 * _NE + i * _NE + e[0], 0, 0))
    tspec = pl.BlockSpec((mb, _HID), lambda mi, e: (0, 0))
    return pl.pallas_call(
        functools.partial(_dual_rope_kernel, rope2=True),
        grid_spec=pltpu.PrefetchScalarGridSpec(
            num_scalar_prefetch=1,
            grid=(m // mb,),
            in_specs=[
                pl.BlockSpec((mb, _HID), lambda mi, e: (mi, 0)),
                wspec(0), wspec(3), bspec(0), bspec(3),
                tspec, tspec, tspec,
            ],
            out_specs=pl.BlockSpec((mb, 2 * _HID), lambda mi, e: (mi, 0)),
        ),
        out_shape=jax.ShapeDtypeStruct((m, 2 * _HID), _BF),
        compiler_params=pltpu.CompilerParams(
            dimension_semantics=("parallel",)),
    )(e_arr, x, w_moe, w_moe, b_moe, b_moe, *qtab)


def _kv_proj(e_arr, kv_in, w_moe, b_moe, ktab, mb):
    wspec = lambda which: pl.BlockSpec(
        (None, None, None, _HID, _HID),
        lambda mi, e: ((mi // 4) % 2, (mi // 8) * 3 + which, e[0], 0, 0))
    bspec = lambda which: pl.BlockSpec(
        (None, 1, _HID),
        lambda mi, e: (((mi // 4) % 2) * 8 * _NE
                       + ((mi // 8) * 3 + which) * _NE + e[0], 0, 0))
    tspec = pl.BlockSpec(
        (None, mb, _HID),
        lambda mi, e: ((mi // 8) * (1 + (mi // 4) % 2), 0, 0))
    return pl.pallas_call(
        functools.partial(_dual_rope_kernel, rope2=False),
        grid_spec=pltpu.PrefetchScalarGridSpec(
            num_scalar_prefetch=1,
            grid=(16,),
            in_specs=[
                pl.BlockSpec((mb, _HID),
                             lambda mi, e: ((mi // 8) * 4 + mi % 4, 0)),
                wspec(1), wspec(2), bspec(1), bspec(2),
                tspec, tspec, tspec,
            ],
            out_specs=pl.BlockSpec((mb, 2 * _HID), lambda mi, e: (mi, 0)),
        ),
        out_shape=jax.ShapeDtypeStruct((16 * mb, 2 * _HID), _BF),
        compiler_params=pltpu.CompilerParams(
            dimension_semantics=("arbitrary",)),
    )(e_arr, kv_in, w_moe, w_moe, b_moe, b_moe, *ktab)


def _oln_ffn_kernel(e_ref, a_ref, wo_ref, bo_ref, res_ref, g_ref, be_ref,
                    wf_ref, bf_ref, o_ref):
    y = jnp.dot(a_ref[...], wo_ref[...].astype(_BF),
                preferred_element_type=_F32) + bo_ref[...] + res_ref[...]
    mu = jnp.mean(y, axis=1, keepdims=True)
    d = y - mu
    var = jnp.mean(d * d, axis=1, keepdims=True)
    hn = (d * jax.lax.rsqrt(var + 1e-5) * g_ref[...] + be_ref[...])
    o_ref[...] = jnp.maximum(
        jnp.dot(hn.astype(_BF), wf_ref[...].astype(_BF),
                preferred_element_type=_F32) + bf_ref[...], 0.0)


def _oln_ffn_head_kernel(e_ref, a_ref, wo_ref, bo_ref, res_ref, g_ref,
                         be_ref, wf_ref, bf_ref, wout_ref, bout_ref, o_ref):
    y = jnp.dot(a_ref[...], wo_ref[...].astype(_BF),
                preferred_element_type=_F32) + bo_ref[...] + res_ref[...]
    mu = jnp.mean(y, axis=1, keepdims=True)
    d = y - mu
    var = jnp.mean(d * d, axis=1, keepdims=True)
    hn = (d * jax.lax.rsqrt(var + 1e-5) * g_ref[...] + be_ref[...])
    t = jnp.maximum(
        jnp.dot(hn.astype(_BF), wf_ref[...].astype(_BF),
                preferred_element_type=_F32) + bf_ref[...], 0.0)
    o_ref[...] = jnp.dot(t.astype(_BF), wout_ref[...],
                         preferred_element_type=_F32) + bout_ref[...]


def _expert_oln_ffn(e_arr, attn, w_moe, b_moe, res, norm_g, norm_b, layer,
                    mb, head=None):
    m = attn.shape[0]
    in_specs = [
        pl.BlockSpec((mb, _HID), lambda mi, e: (mi, 0)),
        pl.BlockSpec((None, None, None, _HID, _HID),
                     lambda mi, e: (layer, 6, e[0], 0, 0)),
        pl.BlockSpec((None, 1, _HID),
                     lambda mi, e: (layer * 8 * _NE + 6 * _NE + e[0], 0, 0)),
        pl.BlockSpec((mb, _HID), lambda mi, e: (mi, 0)),
        pl.BlockSpec((None, 1, _HID),
                     lambda mi, e: (layer * _NE + e[0], 0, 0)),
        pl.BlockSpec((None, 1, _HID),
                     lambda mi, e: (layer * _NE + e[0], 0, 0)),
        pl.BlockSpec((None, None, None, _HID, _HID),
                     lambda mi, e: (layer, 7, e[0], 0, 0)),
        pl.BlockSpec((None, 1, _HID),
                     lambda mi, e: (layer * 8 * _NE + 7 * _NE + e[0], 0, 0)),
    ]
    args = [e_arr, attn, w_moe, b_moe, res, norm_g, norm_b, w_moe, b_moe]
    if head is None:
        kern, n_out, out_dt = _oln_ffn_kernel, _HID, _F32
    else:
        kern, n_out, out_dt = _oln_ffn_head_kernel, 128, _F32
        in_specs += [pl.BlockSpec((_HID, 128), lambda mi, e: (0, 0)),
                     pl.BlockSpec((1, 128), lambda mi, e: (0, 0))]
        args += [head[0], head[1]]
    return pl.pallas_call(
        kern,
        grid_spec=pltpu.PrefetchScalarGridSpec(
            num_scalar_prefetch=1,
            grid=(m // mb,),
            in_specs=in_specs,
            out_specs=pl.BlockSpec((mb, n_out), lambda mi, e: (mi, 0)),
        ),
        out_shape=jax.ShapeDtypeStruct((m, n_out), out_dt),
        compiler_params=pltpu.CompilerParams(
            dimension_semantics=("parallel",)),
    )(*args)


def _attn_kernel(q_ref, kva_ref, kvt_ref, o_ref, nb):
    zero = jnp.zeros((_K, _HD), _BF)
    dn = (((1,), (1,)), ((), ()))
    for j in range(nb):
        rq = slice(j * _T, (j + 1) * _T)
        rk = slice(j * _K, (j + 1) * _K)
        ss = []
        for h in range(_NH):
            sl = slice(h * _HD, (h + 1) * _HD)
            slt = slice(_HID + h * _HD, _HID + (h + 1) * _HD)
            q2 = jnp.concatenate([q_ref[rq, sl], q_ref[rq, slt]], axis=1)
            k2 = jnp.concatenate(
                [jnp.concatenate([kva_ref[rk, sl], zero], axis=1),
                 jnp.concatenate([zero, kvt_ref[rk, sl]], axis=1)], axis=0)
            ss.append(jax.lax.dot_general(q2, k2, dn,
                                          preferred_element_type=_F32))
        for h in range(_NH):
            sl = slice(h * _HD, (h + 1) * _HD)
            slt = slice(_HID + h * _HD, _HID + (h + 1) * _HD)
            s = ss[h]
            mx = jnp.max(s, axis=1, keepdims=True)
            p = jnp.exp(s - mx)
            den = jnp.sum(p, axis=1, keepdims=True)
            w = (p * (1.0 / den)).astype(_BF)
            v2 = jnp.concatenate([kva_ref[rk, slt], kvt_ref[rk, slt]], axis=0)
            o_ref[rq, sl] = jnp.dot(w, v2,
                                    preferred_element_type=_F32).astype(_BF)


def _attention(qq, kv, layer, nb):
    grid = _B // nb
    kb = (nb * _K) // 128
    return pl.pallas_call(
        functools.partial(_attn_kernel, nb=nb),
        grid=(grid,),
        in_specs=[
            pl.BlockSpec((nb * _T, 2 * _HID), lambda i: (i, 0)),
            pl.BlockSpec((nb * _K, 2 * _HID),
                         lambda i: (layer * (16 // kb) + i, 0)),
            pl.BlockSpec((nb * _K, 2 * _HID),
                         lambda i: ((32 + layer * 16) // kb + i, 0)),
        ],
        out_specs=pl.BlockSpec((nb * _T, _HID), lambda i: (i, 0)),
        out_shape=jax.ShapeDtypeStruct((_B * _T, _HID), _BF),
        compiler_params=pltpu.CompilerParams(
            dimension_semantics=("parallel",)),
    )(qq, kv, kv)


def kernel(x, h_a, h_t, W_moe, b_moe, norm_g, norm_b, gate, W_in, b_in,
           W_out, b_out, expert_idx):
    e_arr = jnp.asarray(expert_idx, dtype=jnp.int32).reshape((1,))

    xf = x.reshape(_B * _T, _IN_DIM).astype(_BF)
    kv_in = jnp.concatenate([h_a.reshape(_B * _K, _HID),
                             h_t.reshape(_B * _K, _HID)], axis=0)
    bm3 = b_moe.reshape(_NB * 8 * _NE, 1, _HID)
    ng3 = norm_g.reshape(_NB * _NE, 1, _HID)
    nb3 = norm_b.reshape(_NB * _NE, 1, _HID)

    inv = 1.0 / math.sqrt(_HD)
    qtab = _np_rope_coeff(_T, 512, inv)
    kc, ka, kb_ = _np_rope_coeff(_K, 512, 1.0)
    r0 = jax.nn.sigmoid(gate[0, e_arr[0]])
    r1 = jax.nn.sigmoid(gate[1, e_arr[0]])
    ktab = tuple(jnp.stack([z, z * r0, z * r1]) for z in (kc, ka, kb_))

    h = _matmul_bias(xf, W_in.astype(_BF), b_in.reshape(1, _HID),
                     mb=2048, kb=1024)
    kv = _kv_proj(e_arr, kv_in, W_moe, bm3, ktab, mb=512)

    w_out_p = jnp.zeros((_HID, 128), _F32).at[:, :_ACT].set(W_out).astype(_BF)
    b_out_p = jnp.zeros((1, 128), _F32).at[:, :_ACT].set(b_out.reshape(1, _ACT))

    out = None
    for layer in range(_NB):
        qq = _qq_proj(e_arr, h, W_moe, bm3, qtab, layer, mb=512)
        attn = _attention(qq, kv, layer, nb=8)
        head = None if layer < _NB - 1 else (w_out_p, b_out_p)
        nxt = _expert_oln_ffn(e_arr, attn, W_moe, bm3, h, ng3, nb3, layer,
                              mb=512, head=head)
        if head is None:
            h = nxt
        else:
            out = nxt
    return out[:, :_ACT].reshape(_B, _T, _ACT)
